# baseline (speedup 1.0000x reference)
import jax
import jax.numpy as jnp
from jax.experimental import pallas as pl
from jax.experimental.pallas import tpu as pltpu

_B, _C, _H, _W = 8, 32, 64, 64
_S = _H * _W
_K = 1024
_TOK = 4096
_NT = _S // _TOK
_SL = 8
_NSL = _K // _SL


def _vq_body(z_ref, emb_ref, zq_ref, ids_ref, part_ref,
             scores_ref, oh_ref):
    zb = z_ref[0]
    emb = emb_ref[...]

    scores_ref[...] = jax.lax.dot_general(
        emb * (-2.0), zb, (((1,), (0,)), ((), ())),
        preferred_element_type=jnp.float32)
    e2 = jnp.sum(emb * emb, axis=1, keepdims=True)
    z2 = jnp.sum(zb * zb, axis=0, keepdims=True)

    base8 = jax.lax.broadcasted_iota(
        jnp.int32, (_SL, 1), 0).astype(jnp.float32)

    def dslice(i):
        sl = scores_ref[i * _SL:(i + 1) * _SL, :]
        e2sl = jax.lax.slice(e2, (i * _SL, 0), ((i + 1) * _SL, 1))
        return (z2 + e2sl) + sl

    best = dslice(0)
    bestslice = jnp.zeros((_SL, _TOK), jnp.float32)
    for i in range(1, _NSL):
        dsl = dslice(i)
        m = dsl < best
        best = jnp.where(m, dsl, best)
        bestslice = jnp.where(m, float(i), bestslice)

    bestidx = bestslice * float(_SL) + base8
    dmin = jnp.min(best, axis=0, keepdims=True)
    idsf = jnp.min(jnp.where(best == dmin, bestidx, float(2 * _K)),
                   axis=0, keepdims=True)

    base16 = jax.lax.broadcasted_iota(
        jnp.int32, (16, 1), 0).astype(jnp.float32)
    for j in range(_K // 16):
        kvals = base16 + float(j * 16)
        oh_ref[j * 16:(j + 1) * 16, :] = jnp.where(
            kvals == idsf, 1.0, 0.0).astype(jnp.bfloat16)

    zq = jax.lax.dot_general(
        emb.astype(jnp.bfloat16), oh_ref[...], (((0,), (0,)), ((), ())),
        preferred_element_type=jnp.float32)

    zq_ref[0] = zq
    ids_ref[0, 0] = idsf[0].astype(jnp.int32)

    part_ref[0, 0, 0] = jnp.sum(dmin)


def kernel(z, embedding_table):
    z3 = z.reshape(_B, _C, _S)

    zq3, ids2, parts = pl.pallas_call(
        _vq_body,
        grid=(_B, _NT),
        in_specs=[
            pl.BlockSpec((1, _C, _TOK), lambda b, t: (b, 0, t)),
            pl.BlockSpec((_K, _C), lambda b, t: (0, 0)),
        ],
        out_specs=[
            pl.BlockSpec((1, _C, _TOK), lambda b, t: (b, 0, t)),
            pl.BlockSpec((1, 1, _TOK), lambda b, t: (b * _NT + t, 0, 0)),
            pl.BlockSpec((1, 1, 1), lambda b, t: (b * _NT + t, 0, 0),
                         memory_space=pltpu.SMEM),
        ],
        out_shape=[
            jax.ShapeDtypeStruct((_B, _C, _S), jnp.float32),
            jax.ShapeDtypeStruct((_B * _NT, 1, _TOK), jnp.int32),
            jax.ShapeDtypeStruct((_B * _NT, 1, 1), jnp.float32),
        ],
        scratch_shapes=[
            pltpu.VMEM((_K, _TOK), jnp.float32),
            pltpu.VMEM((_K, _TOK), jnp.bfloat16),
        ],
        compiler_params=pltpu.CompilerParams(
            dimension_semantics=("parallel", "parallel")),
    )(z3, embedding_table)

    zq = zq3.reshape(_B, _C, _H, _W)
    ids = ids2.reshape(_B * _S)
    mse = jnp.sum(parts) * (1.0 / float(_B * _C * _S))
    commitment_loss = 0.25 * mse
    codebook_loss = mse
    loss = commitment_loss + codebook_loss
    return (zq, loss, commitment_loss, codebook_loss, ids)


# speedup vs baseline: 1.1051x; 1.1051x over previous
import jax
import jax.numpy as jnp
from jax.experimental import pallas as pl
from jax.experimental.pallas import tpu as pltpu

_B, _C, _H, _W = 8, 32, 64, 64
_S = _H * _W
_K = 1024
_TOK = 4096
_NT = _S // _TOK
_SL = 8
_NSL = _K // _SL


def _vq_body(z_ref, emb_ref, zq_ref, ids_ref, loss_ref, com_ref, cb_ref,
             scores_ref, oh_ref, acc_ref):
    zb = z_ref[0]
    emb = emb_ref[...]

    scores_ref[...] = jax.lax.dot_general(
        emb * (-2.0), zb, (((1,), (0,)), ((), ())),
        preferred_element_type=jnp.float32)
    e2 = jnp.sum(emb * emb, axis=1, keepdims=True)
    z2 = jnp.sum(zb * zb, axis=0, keepdims=True)

    base8 = jax.lax.broadcasted_iota(
        jnp.int32, (_SL, 1), 0).astype(jnp.float32)

    def dslice(i):
        sl = scores_ref[i * _SL:(i + 1) * _SL, :]
        e2sl = jax.lax.slice(e2, (i * _SL, 0), ((i + 1) * _SL, 1))
        return (z2 + e2sl) + sl

    best = dslice(0)
    bestslice = jnp.zeros((_SL, _TOK), jnp.float32)
    for i in range(1, _NSL):
        dsl = dslice(i)
        m = dsl < best
        best = jnp.where(m, dsl, best)
        bestslice = jnp.where(m, float(i), bestslice)

    bestidx = bestslice * float(_SL) + base8
    dmin = jnp.min(best, axis=0, keepdims=True)
    idsf = jnp.min(jnp.where(best == dmin, bestidx, float(2 * _K)),
                   axis=0, keepdims=True)

    base16 = jax.lax.broadcasted_iota(
        jnp.int32, (16, 1), 0).astype(jnp.float32)
    for j in range(_K // 16):
        kvals = base16 + float(j * 16)
        oh_ref[j * 16:(j + 1) * 16, :] = jnp.where(
            kvals == idsf, 1.0, 0.0).astype(jnp.bfloat16)

    zq = jax.lax.dot_general(
        emb.astype(jnp.bfloat16), oh_ref[...], (((0,), (0,)), ((), ())),
        preferred_element_type=jnp.float32)

    zq_ref[0] = zq
    ids_ref[0, 0] = idsf[0].astype(jnp.int32)

    part = jnp.sum(dmin)

    @pl.when((pl.program_id(0) == 0) & (pl.program_id(1) == 0))
    def _init():
        acc_ref[0, 0] = 0.0

    acc_ref[0, 0] += part

    @pl.when((pl.program_id(0) == _B - 1) & (pl.program_id(1) == _NT - 1))
    def _finalize():
        mse = acc_ref[0, 0] * (1.0 / float(_B * _C * _S))
        com_ref[0, 0] = 0.25 * mse
        cb_ref[0, 0] = mse
        loss_ref[0, 0] = 0.25 * mse + mse


def kernel(z, embedding_table):
    z3 = z.reshape(_B, _C, _S)

    zq3, ids2, loss2, com2, cb2 = pl.pallas_call(
        _vq_body,
        grid=(_B, _NT),
        in_specs=[
            pl.BlockSpec((1, _C, _TOK), lambda b, t: (b, 0, t)),
            pl.BlockSpec((_K, _C), lambda b, t: (0, 0)),
        ],
        out_specs=[
            pl.BlockSpec((1, _C, _TOK), lambda b, t: (b, 0, t)),
            pl.BlockSpec((1, 1, _TOK), lambda b, t: (b * _NT + t, 0, 0)),
            pl.BlockSpec(memory_space=pltpu.SMEM),
            pl.BlockSpec(memory_space=pltpu.SMEM),
            pl.BlockSpec(memory_space=pltpu.SMEM),
        ],
        out_shape=[
            jax.ShapeDtypeStruct((_B, _C, _S), jnp.float32),
            jax.ShapeDtypeStruct((_B * _NT, 1, _TOK), jnp.int32),
            jax.ShapeDtypeStruct((1, 1), jnp.float32),
            jax.ShapeDtypeStruct((1, 1), jnp.float32),
            jax.ShapeDtypeStruct((1, 1), jnp.float32),
        ],
        scratch_shapes=[
            pltpu.VMEM((_K, _TOK), jnp.float32),
            pltpu.VMEM((_K, _TOK), jnp.bfloat16),
            pltpu.SMEM((1, 1), jnp.float32),
        ],
    )(z3, embedding_table)

    zq = zq3.reshape(_B, _C, _H, _W)
    ids = ids2.reshape(_B * _S)
    return (zq, loss2.reshape(()), com2.reshape(()), cb2.reshape(()), ids)
